# baseline (speedup 1.0000x reference)
import functools

import jax
import jax.numpy as jnp
from jax import lax
from jax.experimental import pallas as pl
from jax.experimental.pallas import tpu as pltpu
from jax.experimental.pallas import tpu_sc as plsc

N = 1_600_000
S = 50_000
SP = 51_200
NC = 2
NS = 16
NW = NC * NS
PER_W = N // NW
CHUNK = 10_000
NCHUNK = PER_W // CHUNK
VECS = CHUNK // 16
HALO = 8
MC = 128

_mesh = plsc.VectorSubcoreMesh(core_axis_name="c", subcore_axis_name="s")


def _shift16(x, idx):
    return lax.gather(
        x,
        idx[:, None],
        dimension_numbers=lax.GatherDimensionNumbers(
            offset_dims=(), collapsed_slice_dims=(0,), start_index_map=(0,)
        ),
        slice_sizes=(1,),
        mode=lax.GatherScatterMode.PROMISE_IN_BOUNDS,
    )


@functools.partial(
    pl.kernel,
    out_type=(
        jax.ShapeDtypeStruct((NC * SP,), jnp.float32),
        jax.ShapeDtypeStruct((NC * SP,), jnp.float32),
    ),
    mesh=_mesh,
    compiler_params=pltpu.CompilerParams(needs_layout_passes=False),
    scratch_types=[
        pltpu.VMEM((HALO + CHUNK + HALO,), jnp.int32),
        pltpu.VMEM((CHUNK,), jnp.float32),
        pltpu.VMEM((SP,), jnp.float32),
        pltpu.VMEM((SP,), jnp.float32),
        pltpu.VMEM((16,), jnp.int32),
        pltpu.VMEM((16,), jnp.int32),
        pltpu.VMEM((MC,), jnp.int32),
        pltpu.VMEM_SHARED((SP,), jnp.float32),
        pltpu.VMEM_SHARED((SP,), jnp.float32),
    ],
)
def _seg_kernel(img_hbm, ids_hbm, sum_hbm, cnt_hbm,
                ids_v, val_v, acc_sum, acc_cnt, lo_v, hi_v, idx_v,
                sh_sum, sh_cnt):
    cid = lax.axis_index("c")
    sid = lax.axis_index("s")
    wid = cid * NS + sid
    base = wid * PER_W

    zeros16 = jnp.zeros((16,), jnp.float32)
    iota = lax.iota(jnp.int32, 16)

    sl = SP // NS
    def shz_body(i, carry):
        off = pl.multiple_of(i * 16, 16)
        acc_sum[pl.ds(off, 16)] = zeros16
        return carry
    lax.fori_loop(0, sl // 16, shz_body, 0, unroll=7)
    pltpu.sync_copy(acc_sum.at[pl.ds(0, sl)], sh_sum.at[pl.ds(sid * sl, sl)])
    pltpu.sync_copy(acc_sum.at[pl.ds(0, sl)], sh_cnt.at[pl.ds(sid * sl, sl)])

    pltpu.sync_copy(ids_hbm.at[pl.ds(base, 16)], lo_v)
    pltpu.sync_copy(ids_hbm.at[pl.ds(base + PER_W - 16, 16)], hi_v)
    first = lo_v[pl.ds(0, 16)][0]
    last = hi_v[pl.ds(0, 16)][15]
    zbase = pl.multiple_of(first & ~(MC - 1), 16)
    nz = ((last | (MC - 1)) + 1 - zbase) // 16

    def zero_body(i, carry):
        off = pl.multiple_of(i * 16, 16) + zbase
        acc_sum[pl.ds(off, 16)] = zeros16
        acc_cnt[pl.ds(off, 16)] = zeros16
        return carry

    lax.fori_loop(0, nz, zero_body, 0)

    lane_last = iota == 15
    lane_first = iota == 0
    end_cnt = (iota + 1).astype(jnp.float32)
    start_cnt = -iota.astype(jnp.float32)
    fzero = jnp.zeros((16,), jnp.float32)
    sh_idx = [jnp.maximum(iota - (1 << t), 0) for t in range(4)]
    sh_ok = [iota >= (1 << t) for t in range(4)]

    def _prefix16(v):
        for t in range(4):
            v = v + jnp.where(sh_ok[t], _shift16(v, sh_idx[t]), fzero)
        return v

    def _one_vec(o, use_scan):
        ids = ids_v[pl.ds(o + HALO, 16)]
        vals = val_v[pl.ds(o, 16)]
        ids_next = ids_v[pl.ds(o + HALO + 1, 16)]
        ids_prev = ids_v[pl.ds(o + HALO - 1, 16)]
        is_end = (ids != ids_next) | lane_last
        is_start = (ids != ids_prev) | lane_first
        c = plsc.cumsum(vals) if use_scan else _prefix16(vals)
        nce = vals - c
        plsc.addupdate_scatter(acc_sum, [ids], c, mask=is_end)
        plsc.addupdate_scatter(acc_sum, [ids], nce, mask=is_start)
        plsc.addupdate_scatter(acc_cnt, [ids], end_cnt, mask=is_end)
        plsc.addupdate_scatter(acc_cnt, [ids], start_cnt, mask=is_start)

    def pair_body(k, carry):
        o = pl.multiple_of(k * 32, 32)
        _one_vec(o, True)
        _one_vec(o + 16, False)
        return carry

    def vec_body(k, carry):
        _one_vec(pl.multiple_of(k * 16, 16), True)
        return carry

    def chunk_body(j, carry):
        off = pl.multiple_of(base + j * CHUNK, 8)
        pltpu.sync_copy(ids_hbm.at[pl.ds(off, CHUNK)], ids_v.at[pl.ds(HALO, CHUNK)])
        pltpu.sync_copy(img_hbm.at[pl.ds(off, CHUNK)], val_v)
        lax.fori_loop(0, VECS // 2, pair_body, carry, unroll=4)
        lax.fori_loop(VECS - 1, VECS, vec_body, carry)
        return carry

    lax.fori_loop(0, NCHUNK, chunk_body, 0)

    plsc.subcore_barrier()

    mbase = pl.multiple_of(first & ~(MC - 1), MC)
    nm = (last - mbase) // MC + 1

    def merge_body(m, carry):
        off = pl.multiple_of(m * MC, MC) + mbase
        for k in range(MC // 16):
            idx_v[pl.ds(k * 16, 16)] = iota + (off + k * 16)
        pltpu.sync_copy(acc_sum.at[pl.ds(off, MC)], sh_sum.at[idx_v], add=True)
        pltpu.sync_copy(acc_cnt.at[pl.ds(off, MC)], sh_cnt.at[idx_v], add=True)
        return carry

    lax.fori_loop(0, nm, merge_body, 0)

    plsc.subcore_barrier()

    wo = pl.multiple_of(cid * SP + sid * sl, 8)
    pltpu.sync_copy(sh_sum.at[pl.ds(sid * sl, sl)], sum_hbm.at[pl.ds(wo, sl)])
    pltpu.sync_copy(sh_cnt.at[pl.ds(sid * sl, sl)], cnt_hbm.at[pl.ds(wo, sl)])


def _reduce_body(s0_ref, s1_ref, c0_ref, c1_ref, os_ref, oc_ref):
    os_ref[...] = s0_ref[...] + s1_ref[...]
    oc_ref[...] = c0_ref[...] + c1_ref[...]


_BC = 5120
_NB = SP // _BC


def _tc_reduce(sum_part, cnt_part):
    grid = (S + _BC - 1) // _BC
    return pl.pallas_call(
        _reduce_body,
        grid=(grid,),
        in_specs=[
            pl.BlockSpec((_BC,), lambda i: (i,)),
            pl.BlockSpec((_BC,), lambda i: (i + _NB,)),
            pl.BlockSpec((_BC,), lambda i: (i,)),
            pl.BlockSpec((_BC,), lambda i: (i + _NB,)),
        ],
        out_specs=[
            pl.BlockSpec((_BC,), lambda i: (i,)),
            pl.BlockSpec((_BC,), lambda i: (i,)),
        ],
        out_shape=(
            jax.ShapeDtypeStruct((S,), jnp.float32),
            jax.ShapeDtypeStruct((S,), jnp.float32),
        ),
    )(sum_part, sum_part, cnt_part, cnt_part)


def kernel(image, boxID_ptr):
    ids = boxID_ptr.astype(jnp.int32)
    sum_part, cnt_part = _seg_kernel(image, ids)
    return _tc_reduce(sum_part, cnt_part)


# speedup vs baseline: 2.1477x; 2.1477x over previous
import functools

import jax
import jax.numpy as jnp
from jax import lax
from jax.experimental import pallas as pl
from jax.experimental.pallas import tpu as pltpu
from jax.experimental.pallas import tpu_sc as plsc

N = 1_600_000
S = 50_000
SP = 51_200
NC = 2
NS = 16
NW = NC * NS
PER_W = N // NW
CHUNK = 10_000
NCHUNK = PER_W // CHUNK
VECS = CHUNK // 16
HALO = 8
MC = 128

_mesh = plsc.VectorSubcoreMesh(core_axis_name="c", subcore_axis_name="s")


def _shift16(x, idx):
    return lax.gather(
        x,
        idx[:, None],
        dimension_numbers=lax.GatherDimensionNumbers(
            offset_dims=(), collapsed_slice_dims=(0,), start_index_map=(0,)
        ),
        slice_sizes=(1,),
        mode=lax.GatherScatterMode.PROMISE_IN_BOUNDS,
    )


@functools.partial(
    pl.kernel,
    out_type=(
        jax.ShapeDtypeStruct((NC * SP,), jnp.float32),
        jax.ShapeDtypeStruct((NC * SP,), jnp.float32),
    ),
    mesh=_mesh,
    compiler_params=pltpu.CompilerParams(needs_layout_passes=False),
    scratch_types=[
        pltpu.VMEM((HALO + CHUNK + HALO,), jnp.int32),
        pltpu.VMEM((CHUNK,), jnp.float32),
        pltpu.VMEM((SP,), jnp.float32),
        pltpu.VMEM((SP,), jnp.float32),
        pltpu.VMEM((16,), jnp.int32),
        pltpu.VMEM((16,), jnp.int32),
        pltpu.VMEM((MC,), jnp.int32),
        pltpu.VMEM_SHARED((SP,), jnp.float32),
        pltpu.VMEM_SHARED((SP,), jnp.float32),
    ],
)
def _seg_kernel(img_hbm, ids_hbm, sum_hbm, cnt_hbm,
                ids_v, val_v, acc_sum, acc_cnt, lo_v, hi_v, idx_v,
                sh_sum, sh_cnt):
    cid = lax.axis_index("c")
    sid = lax.axis_index("s")
    wid = cid * NS + sid
    base = wid * PER_W

    zeros16 = jnp.zeros((16,), jnp.float32)
    iota = lax.iota(jnp.int32, 16)

    sl = SP // NS
    def shz_body(i, carry):
        off = pl.multiple_of(i * 16, 16)
        acc_sum[pl.ds(off, 16)] = zeros16
        return carry
    lax.fori_loop(0, sl // 16, shz_body, 0, unroll=7)
    pltpu.sync_copy(acc_sum.at[pl.ds(0, sl)], sh_sum.at[pl.ds(sid * sl, sl)])
    pltpu.sync_copy(acc_sum.at[pl.ds(0, sl)], sh_cnt.at[pl.ds(sid * sl, sl)])

    pltpu.sync_copy(ids_hbm.at[pl.ds(base, 16)], lo_v)
    pltpu.sync_copy(ids_hbm.at[pl.ds(base + PER_W - 16, 16)], hi_v)
    first = lo_v[pl.ds(0, 16)][0]
    last = hi_v[pl.ds(0, 16)][15]
    zbase = pl.multiple_of(first & ~(MC - 1), 16)
    nz = ((last | (MC - 1)) + 1 - zbase) // 16

    def zero_body(i, carry):
        off = pl.multiple_of(i * 16, 16) + zbase
        acc_sum[pl.ds(off, 16)] = zeros16
        acc_cnt[pl.ds(off, 16)] = zeros16
        return carry

    lax.fori_loop(0, nz, zero_body, 0)

    lane_last = iota == 15
    lane_first = iota == 0
    end_cnt = (iota + 1).astype(jnp.float32)
    start_cnt = -iota.astype(jnp.float32)
    fzero = jnp.zeros((16,), jnp.float32)
    sh_idx = [jnp.maximum(iota - (1 << t), 0) for t in range(4)]
    sh_ok = [iota >= (1 << t) for t in range(4)]

    def _prefix16(v):
        for t in range(4):
            v = v + jnp.where(sh_ok[t], _shift16(v, sh_idx[t]), fzero)
        return v

    def _one_vec(o, use_scan):
        ids = ids_v[pl.ds(o + HALO, 16)]
        vals = val_v[pl.ds(o, 16)]
        ids_next = ids_v[pl.ds(o + HALO + 1, 16)]
        ids_prev = ids_v[pl.ds(o + HALO - 1, 16)]
        is_end = (ids != ids_next) | lane_last
        is_start = (ids != ids_prev) | lane_first
        c = plsc.cumsum(vals) if use_scan else _prefix16(vals)
        nce = vals - c
        plsc.addupdate_scatter(acc_sum, [ids], c, mask=is_end)
        plsc.addupdate_scatter(acc_sum, [ids], nce, mask=is_start)
        plsc.addupdate_scatter(acc_cnt, [ids], end_cnt, mask=is_end)
        plsc.addupdate_scatter(acc_cnt, [ids], start_cnt, mask=is_start)

    def chunk_body(j, carry):
        off = pl.multiple_of(base + j * CHUNK, 8)
        pltpu.sync_copy(ids_hbm.at[pl.ds(off, CHUNK)], ids_v.at[pl.ds(HALO, CHUNK)])
        pltpu.sync_copy(img_hbm.at[pl.ds(off, CHUNK)], val_v)

        @functools.partial(plsc.parallel_loop, 0, VECS, unroll=5)
        def _vec_loop(k):
            _one_vec(pl.multiple_of(k * 16, 16), True)

        return carry

    lax.fori_loop(0, NCHUNK, chunk_body, 0)

    plsc.subcore_barrier()

    mbase = pl.multiple_of(first & ~(MC - 1), MC)
    nm = (last - mbase) // MC + 1

    def merge_body(m, carry):
        off = pl.multiple_of(m * MC, MC) + mbase
        for k in range(MC // 16):
            idx_v[pl.ds(k * 16, 16)] = iota + (off + k * 16)
        pltpu.sync_copy(acc_sum.at[pl.ds(off, MC)], sh_sum.at[idx_v], add=True)
        pltpu.sync_copy(acc_cnt.at[pl.ds(off, MC)], sh_cnt.at[idx_v], add=True)
        return carry

    lax.fori_loop(0, nm, merge_body, 0)

    plsc.subcore_barrier()

    wo = pl.multiple_of(cid * SP + sid * sl, 8)
    pltpu.sync_copy(sh_sum.at[pl.ds(sid * sl, sl)], sum_hbm.at[pl.ds(wo, sl)])
    pltpu.sync_copy(sh_cnt.at[pl.ds(sid * sl, sl)], cnt_hbm.at[pl.ds(wo, sl)])


def _reduce_body(s0_ref, s1_ref, c0_ref, c1_ref, os_ref, oc_ref):
    os_ref[...] = s0_ref[...] + s1_ref[...]
    oc_ref[...] = c0_ref[...] + c1_ref[...]


_BC = 5120
_NB = SP // _BC


def _tc_reduce(sum_part, cnt_part):
    grid = (S + _BC - 1) // _BC
    return pl.pallas_call(
        _reduce_body,
        grid=(grid,),
        in_specs=[
            pl.BlockSpec((_BC,), lambda i: (i,)),
            pl.BlockSpec((_BC,), lambda i: (i + _NB,)),
            pl.BlockSpec((_BC,), lambda i: (i,)),
            pl.BlockSpec((_BC,), lambda i: (i + _NB,)),
        ],
        out_specs=[
            pl.BlockSpec((_BC,), lambda i: (i,)),
            pl.BlockSpec((_BC,), lambda i: (i,)),
        ],
        out_shape=(
            jax.ShapeDtypeStruct((S,), jnp.float32),
            jax.ShapeDtypeStruct((S,), jnp.float32),
        ),
    )(sum_part, sum_part, cnt_part, cnt_part)


def kernel(image, boxID_ptr):
    ids = boxID_ptr.astype(jnp.int32)
    sum_part, cnt_part = _seg_kernel(image, ids)
    return _tc_reduce(sum_part, cnt_part)
